# baseline (speedup 1.0000x reference)
import jax
import jax.numpy as jnp
from jax.experimental import pallas as pl
from jax.experimental.pallas import tpu as pltpu

_N = 10000
_C = 128
_ROWS = 50
_B = 250
_NB = _N // _B


def _body(z_hbm, x_hbm, o_hbm, xv, isem, dsem):
    del z_hbm
    ic = pltpu.make_async_copy(x_hbm, xv, isem)
    ic.start()
    ic.wait()
    cs = []
    for b in range(_NB):
        a0 = b * _B
        for j, r0 in enumerate((0, 25)):
            c = pltpu.make_async_copy(
                xv.at[pl.ds(a0, _B), pl.ds(j, 1), :],
                o_hbm.at[pl.ds(a0, _B), pl.ds(r0, 1), :],
                dsem.at[b, j],
            )
            c.start()
            cs.append(c)
    for c in cs:
        c.wait()


def kernel(atom_embeddings):
    z = jnp.full((_N, _ROWS, _C), atom_embeddings[0, 0] * 0.0,
                 atom_embeddings.dtype)
    x3 = atom_embeddings.reshape(_N, 2, _C)
    return pl.pallas_call(
        _body,
        in_specs=[
            pl.BlockSpec(memory_space=pltpu.MemorySpace.HBM),
            pl.BlockSpec(memory_space=pltpu.MemorySpace.HBM),
        ],
        out_specs=pl.BlockSpec(memory_space=pltpu.MemorySpace.HBM),
        out_shape=jax.ShapeDtypeStruct((_N, _ROWS, _C), atom_embeddings.dtype),
        input_output_aliases={0: 0},
        scratch_shapes=[
            pltpu.VMEM((_N, 2, _C), jnp.float32),
            pltpu.SemaphoreType.DMA,
            pltpu.SemaphoreType.DMA((_NB, 2)),
        ],
    )(z, x3)


# speedup vs baseline: 1.0213x; 1.0213x over previous
import jax
import jax.numpy as jnp
from jax.experimental import pallas as pl
from jax.experimental.pallas import tpu as pltpu

_N = 10000
_C = 128
_ROWS = 50
_BA = 500
_NB = _N // _BA


def _body(x_ref, o_ref):
    o_ref[...] = jnp.zeros_like(o_ref)
    o_ref[:, 0:1, :] = x_ref[:, 0:1, :]
    o_ref[:, 25:26, :] = x_ref[:, 1:2, :]


def kernel(atom_embeddings):
    x3 = atom_embeddings.reshape(_N, 2, _C)
    return pl.pallas_call(
        _body,
        grid=(_NB,),
        in_specs=[pl.BlockSpec((_BA, 2, _C), lambda i: (i, 0, 0))],
        out_specs=pl.BlockSpec((_BA, _ROWS, _C), lambda i: (i, 0, 0)),
        out_shape=jax.ShapeDtypeStruct((_N, _ROWS, _C), atom_embeddings.dtype),
    )(x3)


# speedup vs baseline: 1.0263x; 1.0049x over previous
import jax
import jax.numpy as jnp
from jax.experimental import pallas as pl
from jax.experimental.pallas import tpu as pltpu

_N = 10000
_C = 128
_ROWS = 50
_BA = 500
_NB = _N // _BA


def _body(x_ref, o_ref):
    o_ref[...] = jnp.zeros_like(o_ref)
    o_ref[:, 0:1, :] = x_ref[:, 0:1, :]
    o_ref[:, 25:26, :] = x_ref[:, 1:2, :]


def kernel(atom_embeddings):
    x3 = atom_embeddings.reshape(_N, 2, _C)
    return pl.pallas_call(
        _body,
        grid=(_NB,),
        in_specs=[pl.BlockSpec((_BA, 2, _C), lambda i: (i, 0, 0))],
        out_specs=pl.BlockSpec((_BA, _ROWS, _C), lambda i: (i, 0, 0)),
        out_shape=jax.ShapeDtypeStruct((_N, _ROWS, _C), atom_embeddings.dtype),
        compiler_params=pltpu.CompilerParams(
            dimension_semantics=("parallel",),
        ),
    )(x3)
